# Initial kernel scaffold; baseline (speedup 1.0000x reference)
import jax
import jax.numpy as jnp
from jax.experimental import pallas as pl


def kernel(x_s, edge_index_s, edge_weight_s, edge_index_s1, edge_weight_s1, W0, b0, g0, be0, W1, b1, g1, be1, W2, b2, g2, be2, lin1_W, lin1_b, bn1_g, bn1_b, lin2_W, lin2_b, bn2_g, bn2_b, lin3_W, lin3_b):
    raise NotImplementedError("write your pallas kernel here")


# speedup vs baseline: 17.9047x; 17.9047x over previous
import functools

import jax
import jax.numpy as jnp
from jax import lax
from jax.experimental import pallas as pl
from jax.experimental.pallas import tpu as pltpu
from jax.experimental.pallas import tpu_sc as plsc

EDGE_NUM = 8978
N0 = 16 * EDGE_NUM
N1 = N0 // 2
LEAKY = 0.33

NSUB = 16
NCORE = 2
ROWS = 568
G = 8
N0P = 145408
PT1 = 4544
N1P = NSUB * PT1
B1 = 64

NX0 = 35968
NX1 = 17984
BK = 2248
F32 = jnp.float32


def _mesh():
    return plsc.VectorSubcoreMesh(core_axis_name="c", subcore_axis_name="s",
                                  num_cores=NCORE, num_subcores=NSUB)


def _make_sc_prop_c1():
    @functools.partial(
        pl.kernel,
        out_type=jax.ShapeDtypeStruct((NCORE, N0P), F32),
        mesh=_mesh(),
        compiler_params=pltpu.CompilerParams(use_tc_tiling_on_sc=False),
        scratch_types=[
            pltpu.VMEM((G, 128), jnp.int32),
            pltpu.VMEM((G, 128), jnp.int32),
            pltpu.VMEM((G, 128), F32),
            pltpu.VMEM((G, 128), F32),
            pltpu.VMEM((9088,), F32),
            pltpu.VMEM_SHARED((N0P,), F32),
            pltpu.SemaphoreType.DMA,
        ],
    )
    def k(src_r, dst_r, ew_r, x_r, out_r, sbuf, dbuf, wbuf, vbuf, zbuf, acc, sem):
        c = lax.axis_index("c")
        s = lax.axis_index("s")
        w = c * NSUB + s

        def zb(i, _):
            zbuf[pl.ds(i * 16, 16)] = jnp.zeros((16,), F32)
            return 0
        lax.fori_loop(0, 568, zb, 0)
        pltpu.sync_copy(zbuf, acc.at[pl.ds(s * 9088, 9088)])
        plsc.subcore_barrier()

        def grp(g, _):
            pltpu.sync_copy(src_r.at[w].at[pl.ds(g * G, G)], sbuf)
            pltpu.sync_copy(dst_r.at[w].at[pl.ds(g * G, G)], dbuf)
            pltpu.sync_copy(ew_r.at[w].at[pl.ds(g * G, G)], wbuf)
            for j in range(G):
                pltpu.async_copy(x_r.at[sbuf.at[j]], vbuf.at[j], sem).wait()
                for q in range(8):
                    sl = pl.ds(q * 16, 16)
                    vbuf[j, sl] = vbuf[j, sl] * wbuf[j, sl]
                pltpu.sync_copy(vbuf.at[j], acc.at[dbuf.at[j]], add=True)
            return 0
        lax.fori_loop(0, ROWS // G, grp, 0)
        plsc.subcore_barrier()
        pltpu.sync_copy(acc.at[pl.ds(s * 9088, 9088)],
                        out_r.at[c].at[pl.ds(s * 9088, 9088)])

    return k


def _make_sc_prop_c32(alpha, beta, gamma):
    @functools.partial(
        pl.kernel,
        out_type=jax.ShapeDtypeStruct((NCORE, N1P, 16), F32),
        mesh=_mesh(),
        compiler_params=pltpu.CompilerParams(use_tc_tiling_on_sc=False),
        scratch_types=[
            pltpu.VMEM((G, 128), jnp.int32),
            pltpu.VMEM((G, 128), jnp.int32),
            pltpu.VMEM((G, 128), F32),
            pltpu.VMEM((G, 128, 16), F32),
            pltpu.VMEM((B1, 16), F32),
            pltpu.VMEM((B1, 16), F32),
            pltpu.VMEM((B1, 16), F32),
            pltpu.VMEM((B1, 16), F32),
            pltpu.VMEM_SHARED((N1P, 16), F32),
            pltpu.SemaphoreType.DMA,
        ],
    )
    def k(src_r, dst_r, ew_r, x_r, prev_r, out_r,
          sbuf, dbuf, wbuf, vbuf, abuf, xbuf, pbuf, ybuf, acc, sem):
        c = lax.axis_index("c")
        s = lax.axis_index("s")

        for i in range(B1):
            abuf[i, :] = jnp.zeros((16,), F32)

        def zb(t, _):
            pltpu.sync_copy(abuf, acc.at[pl.ds(s * PT1 + t * B1, B1)])
            return 0
        lax.fori_loop(0, PT1 // B1, zb, 0)
        plsc.subcore_barrier()

        xc = x_r.at[c]

        def grp(g, _):
            pltpu.sync_copy(src_r.at[s].at[pl.ds(g * G, G)], sbuf)
            pltpu.sync_copy(dst_r.at[s].at[pl.ds(g * G, G)], dbuf)
            pltpu.sync_copy(ew_r.at[s].at[pl.ds(g * G, G)], wbuf)
            for j in range(G):
                pltpu.async_copy(xc.at[sbuf.at[j]], vbuf.at[j], sem).wait()

                def mrow(i, _):
                    base = i * 16
                    wv = wbuf[j, pl.ds(base, 16)]
                    for u in range(16):
                        vbuf[j, base + u, :] = vbuf[j, base + u, :] * wv[u]
                    return 0
                lax.fori_loop(0, 8, mrow, 0)
                pltpu.sync_copy(vbuf.at[j], acc.at[dbuf.at[j]], add=True)
            return 0
        lax.fori_loop(0, ROWS // G, grp, 0)
        plsc.subcore_barrier()

        def ep(t, _):
            r0 = s * PT1 + t * B1
            pltpu.sync_copy(acc.at[pl.ds(r0, B1)], abuf)
            pltpu.sync_copy(x_r.at[c].at[pl.ds(r0, B1)], xbuf)
            if gamma != 0.0:
                pltpu.sync_copy(prev_r.at[c].at[pl.ds(r0, B1)], pbuf)
            for i in range(B1):
                y = alpha * abuf[i, :] + beta * xbuf[i, :]
                if gamma != 0.0:
                    y = y + gamma * pbuf[i, :]
                ybuf[i, :] = y
            pltpu.sync_copy(ybuf, out_r.at[c].at[pl.ds(r0, B1)])
            return 0
        lax.fori_loop(0, PT1 // B1, ep, 0)

    return k


_SC_PROP_C1 = _make_sc_prop_c1()
_SC_PROP_C32_A = _make_sc_prop_c32(1.0, -1.0, 0.0)
_SC_PROP_C32_B = _make_sc_prop_c32(2.0, -2.0, -1.0)


def _leaky(h):
    return jnp.where(h >= 0, h, LEAKY * h)


def _p32():
    ri = lax.broadcasted_iota(jnp.int32, (128, 128), 0)
    ci = lax.broadcasted_iota(jnp.int32, (128, 128), 1)
    return jnp.where(ri % 32 == ci % 32, 0.25, 0.0).astype(F32)


def _full_spec(shape):
    return pl.BlockSpec(shape, lambda i: (0,) * len(shape))


def _acc_spec():
    return pl.BlockSpec((8, 128), lambda i: (0, 0))


def _axpy_body(alpha, beta, gamma, p_ref, x_ref, prev_ref, y_ref):
    p = p_ref[...]
    y = alpha * (p[0] + p[1]) + beta * x_ref[...]
    if gamma != 0.0:
        y = y + gamma * prev_ref[...]
    y_ref[...] = y


def _tc_axpy(p, x, prev, alpha, beta, gamma):
    body = functools.partial(_axpy_body, alpha, beta, gamma)
    return pl.pallas_call(
        body,
        out_shape=jax.ShapeDtypeStruct((N0P // 128, 128), F32),
    )(p.reshape(2, N0P // 128, 128), x, prev)


def _wb0(w_ref):
    w4 = w_ref[...][:, 0, :]
    wt = jnp.concatenate([w4, w4, w4, w4], axis=1)
    wt = jnp.concatenate([wt, wt, wt, wt], axis=0)
    ri = lax.broadcasted_iota(jnp.int32, (16, 128), 0)
    ci = lax.broadcasted_iota(jnp.int32, (16, 128), 1)
    return jnp.where(ri // 4 == ci // 32, wt, 0.0)


def _d0p1_body(a_ref, w_ref, b_ref, pool_ref, s1_ref, s2_ref):
    i = pl.program_id(0)
    b128 = jnp.concatenate([b_ref[...]] * 4)
    h = jnp.dot(a_ref[...], _wb0(w_ref), preferred_element_type=F32,
                precision=lax.Precision.HIGHEST) \
        + b128[None, :]
    s1 = jnp.sum(h, axis=0)
    s2 = jnp.sum(h * h, axis=0)

    @pl.when(i == 0)
    def _():
        s1_ref[...] = jnp.zeros((8, 128), F32)
        s2_ref[...] = jnp.zeros((8, 128), F32)

    s1_ref[...] += jnp.broadcast_to(s1[None, :], (8, 128)) * 0.125
    s2_ref[...] += jnp.broadcast_to(s2[None, :], (8, 128)) * 0.125
    pool_ref[...] = jnp.concatenate(
        [jnp.maximum(h[:, 0:32], h[:, 32:64]),
         jnp.maximum(h[:, 64:96], h[:, 96:128])], axis=1)


def _d0p2_body(pool_ref, s1_ref, s2_ref, b_ref, g_ref, be_ref, out_ref):
    b128 = jnp.concatenate([b_ref[...]] * 4)[None, :]
    s1 = jnp.sum(s1_ref[...], axis=0, keepdims=True)
    s2 = jnp.sum(s2_ref[...], axis=0, keepdims=True)
    npad = NX0 - N0 // 4
    m128 = (s1 - npad * b128) / (N0 // 4)
    e128 = (s2 - npad * b128 * b128) / (N0 // 4)
    p = _p32()
    mc = jnp.dot(m128, p, preferred_element_type=F32,
                precision=lax.Precision.HIGHEST)
    e2c = jnp.dot(e128, p, preferred_element_type=F32,
                precision=lax.Precision.HIGHEST)
    sd = jnp.sqrt(e2c - mc * mc + 1e-5)
    g64 = jnp.concatenate([g_ref[...]] * 2)[None, :]
    be64 = jnp.concatenate([be_ref[...]] * 2)[None, :]
    h = g64 * (pool_ref[...] - mc[:, :64]) / sd[:, :64] + be64
    out_ref[...] = _leaky(h)


def _tc_dense0(a, W0, b0, g0, be0):
    grid = NX0 // BK
    pooled, s1, s2 = pl.pallas_call(
        _d0p1_body,
        grid=(grid,),
        in_specs=[pl.BlockSpec((BK, 16), lambda i: (i, 0)),
                  _full_spec((4, 1, 32)), _full_spec((32,))],
        out_specs=[pl.BlockSpec((BK, 64), lambda i: (i, 0)),
                   _acc_spec(), _acc_spec()],
        out_shape=[jax.ShapeDtypeStruct((NX0, 64), F32),
                   jax.ShapeDtypeStruct((8, 128), F32),
                   jax.ShapeDtypeStruct((8, 128), F32)],
    )(a, W0, b0)
    return pl.pallas_call(
        _d0p2_body,
        grid=(grid,),
        in_specs=[pl.BlockSpec((BK, 64), lambda i: (i, 0)),
                  _acc_spec(), _acc_spec(),
                  _full_spec((32,)), _full_spec((32,)), _full_spec((32,))],
        out_specs=pl.BlockSpec((BK, 64), lambda i: (i, 0)),
        out_shape=jax.ShapeDtypeStruct((NX0, 64), F32),
    )(pooled, s1, s2, b0, g0, be0)


def _bd32(w):
    wt = jnp.concatenate([w, w, w, w], axis=1)
    wt = jnp.concatenate([wt, wt, wt, wt], axis=0)
    ri = lax.broadcasted_iota(jnp.int32, (128, 128), 0)
    ci = lax.broadcasted_iota(jnp.int32, (128, 128), 1)
    return jnp.where(ri // 32 == ci // 32, wt, 0.0)


def _mid_h(x_ref, t1_ref, t2_ref, t3_ref, w_ref, b_ref):
    w = w_ref[...]
    h = jnp.dot(x_ref[...], _bd32(w[0]), preferred_element_type=F32,
                precision=lax.Precision.HIGHEST)
    h += jnp.dot(t1_ref[...], _bd32(w[1]), preferred_element_type=F32,
                precision=lax.Precision.HIGHEST)
    h += jnp.dot(t2_ref[...], _bd32(w[2]), preferred_element_type=F32,
                precision=lax.Precision.HIGHEST)
    h += jnp.dot(t3_ref[...], _bd32(w[3]), preferred_element_type=F32,
                precision=lax.Precision.HIGHEST)
    return h + jnp.concatenate([b_ref[...]] * 4)[None, :]


def _midp1_body(x_ref, t1_ref, t2_ref, t3_ref, w_ref, b_ref, s1_ref, s2_ref):
    i = pl.program_id(0)
    h = _mid_h(x_ref, t1_ref, t2_ref, t3_ref, w_ref, b_ref)

    @pl.when(i == 0)
    def _():
        s1_ref[...] = jnp.zeros((8, 128), F32)
        s2_ref[...] = jnp.zeros((8, 128), F32)

    s1_ref[...] += jnp.broadcast_to(jnp.sum(h, axis=0)[None, :], (8, 128)) * 0.125
    s2_ref[...] += jnp.broadcast_to(jnp.sum(h * h, axis=0)[None, :], (8, 128)) * 0.125


def _midp2_body(x_ref, t1_ref, t2_ref, t3_ref, w_ref, b_ref,
                s1_ref, s2_ref, g_ref, be_ref, out_ref):
    h = _mid_h(x_ref, t1_ref, t2_ref, t3_ref, w_ref, b_ref)
    b128 = jnp.concatenate([b_ref[...]] * 4)[None, :]
    s1 = jnp.sum(s1_ref[...], axis=0, keepdims=True)
    s2 = jnp.sum(s2_ref[...], axis=0, keepdims=True)
    npad = NX1 - N1 // 4
    m128 = (s1 - npad * b128) / (N1 // 4)
    e128 = (s2 - npad * b128 * b128) / (N1 // 4)
    p = _p32()
    mc = jnp.dot(m128, p, preferred_element_type=F32,
                precision=lax.Precision.HIGHEST)
    e2c = jnp.dot(e128, p, preferred_element_type=F32,
                precision=lax.Precision.HIGHEST)
    sd = jnp.sqrt(e2c - mc * mc + 1e-5)
    g128 = jnp.concatenate([g_ref[...]] * 4)[None, :]
    be128 = jnp.concatenate([be_ref[...]] * 4)[None, :]
    out = _leaky(g128 * (h - mc) / sd + be128)
    row = pl.program_id(0) * BK + lax.broadcasted_iota(jnp.int32, out.shape, 0)
    out_ref[...] = jnp.where(row < N1 // 4, out, 0.0)


def _tc_dense_mid(x, t1, t2, t3, W1, b1, g1, be1):
    grid = NX1 // BK
    blk = pl.BlockSpec((BK, 128), lambda i: (i, 0))
    s1, s2 = pl.pallas_call(
        _midp1_body,
        grid=(grid,),
        in_specs=[blk, blk, blk, blk,
                  _full_spec((4, 32, 32)), _full_spec((32,))],
        out_specs=[_acc_spec(), _acc_spec()],
        out_shape=[jax.ShapeDtypeStruct((8, 128), F32),
                   jax.ShapeDtypeStruct((8, 128), F32)],
    )(x, t1, t2, t3, W1, b1)
    return pl.pallas_call(
        _midp2_body,
        grid=(grid,),
        in_specs=[blk, blk, blk, blk,
                  _full_spec((4, 32, 32)), _full_spec((32,)),
                  _acc_spec(), _acc_spec(),
                  _full_spec((32,)), _full_spec((32,))],
        out_specs=blk,
        out_shape=jax.ShapeDtypeStruct((NX1, 128), F32),
    )(x, t1, t2, t3, W1, b1, s1, s2, g1, be1)


def _tail_v(x_ref, u1_ref, u2_ref, u3_ref, w_ref, b_ref):
    w = w_ref[...]

    def t128(k):
        return jnp.concatenate([w[k, :, 0]] * 4)[None, :]

    q = (x_ref[...] * t128(0) + u1_ref[...] * t128(1)
         + u2_ref[...] * t128(2) + u3_ref[...] * t128(3))
    ri = lax.broadcasted_iota(jnp.int32, (128, 4), 0)
    ci = lax.broadcasted_iota(jnp.int32, (128, 4), 1)
    sel = jnp.where(ri // 32 == ci, 1.0, 0.0).astype(F32)
    return jnp.dot(q, sel, preferred_element_type=F32,
                precision=lax.Precision.HIGHEST) + b_ref[...][0]


def _tailp1_body(x_ref, u1_ref, u2_ref, u3_ref, w_ref, b_ref,
                 v_ref, s1_ref, s2_ref):
    i = pl.program_id(0)
    v = _tail_v(x_ref, u1_ref, u2_ref, u3_ref, w_ref, b_ref)

    @pl.when(i == 0)
    def _():
        s1_ref[...] = jnp.zeros((8, 128), F32)
        s2_ref[...] = jnp.zeros((8, 128), F32)

    s1_ref[...] += jnp.full((8, 128), jnp.sum(v) / 1024.0, F32)
    s2_ref[...] += jnp.full((8, 128), jnp.sum(v * v) / 1024.0, F32)
    v_ref[...] = v


def _tc_tail_p1(x, u1, u2, u3, W2, b2):
    grid = NX1 // BK
    blk = pl.BlockSpec((BK, 128), lambda i: (i, 0))
    return pl.pallas_call(
        _tailp1_body,
        grid=(grid,),
        in_specs=[blk, blk, blk, blk,
                  _full_spec((4, 32, 1)), _full_spec((1,))],
        out_specs=[pl.BlockSpec((BK, 4), lambda i: (i, 0)),
                   _acc_spec(), _acc_spec()],
        out_shape=[jax.ShapeDtypeStruct((NX1, 4), F32),
                   jax.ShapeDtypeStruct((8, 128), F32),
                   jax.ShapeDtypeStruct((8, 128), F32)],
    )(x, u1, u2, u3, W2, b2)


def _mlp_body(z_ref, s1_ref, s2_ref, b_ref, g_ref, be_ref,
              l1w_ref, l1b_ref, n1g_ref, n1b_ref,
              l2w_ref, l2b_ref, n2g_ref, n2b_ref,
              l3w_ref, l3b_ref, out_ref):
    npad = (NX1 - N1 // 4) * 4
    b2 = b_ref[...][0]
    sv = jnp.sum(s1_ref[...]) - npad * b2
    sq = jnp.sum(s2_ref[...]) - npad * b2 * b2
    m = sv / N1
    var = sq / N1 - m * m
    z = g_ref[...][0] * (z_ref[...] - m) / jnp.sqrt(var + 1e-5) + be_ref[...][0]
    z = _leaky(z)
    z = jnp.dot(z, l1w_ref[...], preferred_element_type=F32,
                precision=lax.Precision.HIGHEST) + l1b_ref[...][None, :]
    m1 = jnp.mean(z, axis=0)
    d1 = z - m1[None, :]
    v1 = jnp.mean(d1 * d1, axis=0)
    z = n1g_ref[...][None, :] * d1 / jnp.sqrt(v1 + 1e-5)[None, :] + n1b_ref[...][None, :]
    z = jnp.maximum(z, 0.0)
    z = jnp.dot(z, l2w_ref[...], preferred_element_type=F32,
                precision=lax.Precision.HIGHEST) + l2b_ref[...][None, :]
    m2 = jnp.mean(z, axis=0)
    d2 = z - m2[None, :]
    v2 = jnp.mean(d2 * d2, axis=0)
    z = n2g_ref[...][None, :] * d2 / jnp.sqrt(v2 + 1e-5)[None, :] + n2b_ref[...][None, :]
    z = jnp.maximum(z, 0.0)
    out_ref[...] = jnp.dot(z, l3w_ref[...], preferred_element_type=F32,
                precision=lax.Precision.HIGHEST) \
        + l3b_ref[...][None, :]


def _tc_mlp(z, s1, s2, b2, g2, be2, lin1_W, lin1_b, bn1_g, bn1_b,
            lin2_W, lin2_b, bn2_g, bn2_b, lin3_W, lin3_b):
    return pl.pallas_call(
        _mlp_body,
        out_shape=jax.ShapeDtypeStruct((16, 1), F32),
    )(z, s1, s2, b2, g2, be2, lin1_W, lin1_b, bn1_g, bn1_b,
      lin2_W, lin2_b, bn2_g, bn2_b, lin3_W, lin3_b)


def _prep_edges(ei, ew, nt):
    cap = nt * ROWS * 128
    src = jnp.pad(ei[0].astype(jnp.int32), (0, cap - ew.shape[0]))
    dst = jnp.pad(ei[1].astype(jnp.int32), (0, cap - ew.shape[0]))
    w = jnp.pad(ew, (0, cap - ew.shape[0]))
    return (src.reshape(nt, ROWS, 128), dst.reshape(nt, ROWS, 128),
            w.reshape(nt, ROWS, 128))


def _to_sc(flat):
    halves = jnp.stack([flat[:, :16], flat[:, 16:]])
    return jnp.pad(halves, ((0, 0), (0, N1P - N1), (0, 0)))


def _to_x4(sc):
    flat = jnp.concatenate([sc[0, :N1], sc[1, :N1]], axis=1)
    return jnp.pad(flat.reshape(N1 // 4, 128), ((0, NX1 - N1 // 4), (0, 0)))


def kernel(x_s, edge_index_s, edge_weight_s, edge_index_s1, edge_weight_s1,
           W0, b0, g0, be0, W1, b1, g1, be1, W2, b2, g2, be2,
           lin1_W, lin1_b, bn1_g, bn1_b, lin2_W, lin2_b, bn2_g, bn2_b,
           lin3_W, lin3_b):
    src0, dst0, ew0 = _prep_edges(edge_index_s, edge_weight_s, NCORE * NSUB)
    src1, dst1, ew1 = _prep_edges(edge_index_s1, edge_weight_s1, NSUB)

    x0f = jnp.pad(x_s[:, 0], (0, N0P - N0))
    x0p = x0f.reshape(N0P // 128, 128)

    p1 = _SC_PROP_C1(src0, dst0, ew0, x0f)
    tx1 = _tc_axpy(p1, x0p, x0p, 1.0, -1.0, 0.0)
    p2 = _SC_PROP_C1(src0, dst0, ew0, tx1.reshape(N0P))
    tx2 = _tc_axpy(p2, tx1, x0p, 2.0, -2.0, -1.0)
    p3 = _SC_PROP_C1(src0, dst0, ew0, tx2.reshape(N0P))
    tx3 = _tc_axpy(p3, tx2, tx1, 2.0, -2.0, -1.0)

    stack = jnp.stack([x0f[:N0], tx1.reshape(N0P)[:N0],
                       tx2.reshape(N0P)[:N0], tx3.reshape(N0P)[:N0]], axis=1)
    a = jnp.pad(stack.reshape(N0 // 4, 16), ((0, NX0 - N0 // 4), (0, 0)))
    pooled = _tc_dense0(a, W0, b0, g0, be0)

    x1_flat = pooled[:N0 // 4].reshape(N1, 32)
    x1_sc = _to_sc(x1_flat)
    x1_x4 = jnp.pad(x1_flat.reshape(N1 // 4, 128), ((0, NX1 - N1 // 4), (0, 0)))

    t1 = _SC_PROP_C32_A(src1, dst1, ew1, x1_sc, x1_sc)
    t2 = _SC_PROP_C32_B(src1, dst1, ew1, t1, x1_sc)
    t3 = _SC_PROP_C32_B(src1, dst1, ew1, t2, t1)
    x2_x4 = _tc_dense_mid(x1_x4, _to_x4(t1), _to_x4(t2), _to_x4(t3),
                          W1, b1, g1, be1)
    x2_sc = _to_sc(x2_x4[:N1 // 4].reshape(N1, 32))

    u1 = _SC_PROP_C32_A(src1, dst1, ew1, x2_sc, x2_sc)
    u2 = _SC_PROP_C32_B(src1, dst1, ew1, u1, x2_sc)
    u3 = _SC_PROP_C32_B(src1, dst1, ew1, u2, u1)

    v4, s1, s2 = _tc_tail_p1(x2_x4, _to_x4(u1), _to_x4(u2), _to_x4(u3), W2, b2)
    z = v4[:N1 // 4].reshape(16, EDGE_NUM // 2)
    return _tc_mlp(z, s1, s2, b2, g2, be2, lin1_W, lin1_b, bn1_g, bn1_b,
                   lin2_W, lin2_b, bn2_g, bn2_b, lin3_W, lin3_b)


# speedup vs baseline: 27.6297x; 1.5432x over previous
import functools

import jax
import jax.numpy as jnp
from jax import lax
from jax.experimental import pallas as pl
from jax.experimental.pallas import tpu as pltpu
from jax.experimental.pallas import tpu_sc as plsc

EDGE_NUM = 8978
N0 = 16 * EDGE_NUM
N1 = N0 // 2
LEAKY = 0.33

NSUB = 16
NCORE = 2
ROWS = 568
G = 8
N0P = 145408
PT1 = 4544
N1P = NSUB * PT1
EB = 568

NX0 = 35968
NX1 = 17984
BK = 2248
F32 = jnp.float32


def _mesh():
    return plsc.VectorSubcoreMesh(core_axis_name="c", subcore_axis_name="s",
                                  num_cores=NCORE, num_subcores=NSUB)


def _make_sc_prop_c1():
    @functools.partial(
        pl.kernel,
        out_type=jax.ShapeDtypeStruct((NCORE, N0P), F32),
        mesh=_mesh(),
        compiler_params=pltpu.CompilerParams(use_tc_tiling_on_sc=False),
        scratch_types=[
            pltpu.VMEM((G, 128), jnp.int32),
            pltpu.VMEM((G, 128), jnp.int32),
            pltpu.VMEM((G, 128), F32),
            pltpu.VMEM((G, 128), F32),
            pltpu.VMEM((9088,), F32),
            pltpu.VMEM_SHARED((N0P,), F32),
            pltpu.SemaphoreType.DMA,
        ],
    )
    def k(src_r, dst_r, ew_r, x_r, out_r, sbuf, dbuf, wbuf, vbuf, zbuf, acc, sem):
        c = lax.axis_index("c")
        s = lax.axis_index("s")
        w = c * NSUB + s

        def zb(i, _):
            zbuf[pl.ds(i * 16, 16)] = jnp.zeros((16,), F32)
            return 0
        lax.fori_loop(0, 568, zb, 0)
        pltpu.sync_copy(zbuf, acc.at[pl.ds(s * 9088, 9088)])
        plsc.subcore_barrier()

        def grp(g, _):
            pltpu.sync_copy(src_r.at[w].at[pl.ds(g * G, G)], sbuf)
            pltpu.sync_copy(dst_r.at[w].at[pl.ds(g * G, G)], dbuf)
            pltpu.sync_copy(ew_r.at[w].at[pl.ds(g * G, G)], wbuf)
            gd = [pltpu.async_copy(x_r.at[sbuf.at[j]], vbuf.at[j], sem)
                  for j in range(G)]
            for j in range(G):
                gd[j].wait()
                for q in range(8):
                    sl = pl.ds(q * 16, 16)
                    vbuf[j, sl] = vbuf[j, sl] * wbuf[j, sl]
            for j in range(G):
                pltpu.sync_copy(vbuf.at[j], acc.at[dbuf.at[j]], add=True)
            return 0
        lax.fori_loop(0, ROWS // G, grp, 0)
        plsc.subcore_barrier()
        pltpu.sync_copy(acc.at[pl.ds(s * 9088, 9088)],
                        out_r.at[c].at[pl.ds(s * 9088, 9088)])

    return k


def _make_sc_prop_c32(alpha, beta, gamma):
    @functools.partial(
        pl.kernel,
        out_type=jax.ShapeDtypeStruct((NCORE, N1P, 16), F32),
        mesh=_mesh(),
        compiler_params=pltpu.CompilerParams(use_tc_tiling_on_sc=False),
        scratch_types=[
            pltpu.VMEM((G, 128), jnp.int32),
            pltpu.VMEM((G, 128), jnp.int32),
            pltpu.VMEM((G, 128), F32),
            pltpu.VMEM((G, 128, 16), F32),
            pltpu.VMEM((EB, 16), F32),
            pltpu.VMEM((EB, 16), F32),
            pltpu.VMEM((EB, 16), F32),
            pltpu.VMEM((EB, 16), F32),
            pltpu.VMEM_SHARED((N1P, 16), F32),
            pltpu.SemaphoreType.DMA,
        ],
    )
    def k(src_r, dst_r, ew_r, x_r, prev_r, out_r,
          sbuf, dbuf, wbuf, vbuf, abuf, xbuf, pbuf, ybuf, acc, sem):
        c = lax.axis_index("c")
        s = lax.axis_index("s")

        def zrow(i, _):
            ybuf[i, :] = jnp.zeros((16,), F32)
            return 0
        lax.fori_loop(0, EB, zrow, 0)

        def zb(t, _):
            pltpu.sync_copy(ybuf, acc.at[pl.ds(s * PT1 + t * EB, EB)])
            return 0
        lax.fori_loop(0, PT1 // EB, zb, 0)
        plsc.subcore_barrier()

        xc = x_r.at[c]

        def grp(g, _):
            pltpu.sync_copy(src_r.at[s].at[pl.ds(g * G, G)], sbuf)
            pltpu.sync_copy(dst_r.at[s].at[pl.ds(g * G, G)], dbuf)
            pltpu.sync_copy(ew_r.at[s].at[pl.ds(g * G, G)], wbuf)
            gd = [pltpu.async_copy(xc.at[sbuf.at[j]], vbuf.at[j], sem)
                  for j in range(G)]
            for j in range(G):
                gd[j].wait()

                def mrow(i, _):
                    base = i * 16
                    wv = wbuf[j, pl.ds(base, 16)]
                    for u in range(16):
                        vbuf[j, base + u, :] = vbuf[j, base + u, :] * wv[u]
                    return 0
                lax.fori_loop(0, 8, mrow, 0)
            for j in range(G):
                pltpu.sync_copy(vbuf.at[j], acc.at[dbuf.at[j]], add=True)
            return 0
        lax.fori_loop(0, ROWS // G, grp, 0)
        plsc.subcore_barrier()

        for t in range(PT1 // EB):
            r0 = s * PT1 + t * EB
            pltpu.sync_copy(acc.at[pl.ds(r0, EB)], abuf)
            pltpu.sync_copy(x_r.at[c].at[pl.ds(r0, EB)], xbuf)
            if gamma != 0.0:
                pltpu.sync_copy(prev_r.at[c].at[pl.ds(r0, EB)], pbuf)

            def erow(i, _):
                for u in range(8):
                    r = i * 8 + u
                    y = alpha * abuf[r, :] + beta * xbuf[r, :]
                    if gamma != 0.0:
                        y = y + gamma * pbuf[r, :]
                    ybuf[r, :] = y
                return 0
            lax.fori_loop(0, EB // 8, erow, 0)
            pltpu.sync_copy(ybuf, out_r.at[c].at[pl.ds(r0, EB)])

    return k


_SC_PROP_C1 = _make_sc_prop_c1()
_SC_PROP_C32_A = _make_sc_prop_c32(1.0, -1.0, 0.0)
_SC_PROP_C32_B = _make_sc_prop_c32(2.0, -2.0, -1.0)


def _leaky(h):
    return jnp.where(h >= 0, h, LEAKY * h)


def _p32():
    ri = lax.broadcasted_iota(jnp.int32, (128, 128), 0)
    ci = lax.broadcasted_iota(jnp.int32, (128, 128), 1)
    return jnp.where(ri % 32 == ci % 32, 0.25, 0.0).astype(F32)


def _full_spec(shape):
    return pl.BlockSpec(shape, lambda i: (0,) * len(shape))


def _acc_spec():
    return pl.BlockSpec((8, 128), lambda i: (0, 0))


def _axpy_body(alpha, beta, gamma, p_ref, x_ref, prev_ref, y_ref):
    p = p_ref[...]
    y = alpha * (p[0] + p[1]) + beta * x_ref[...]
    if gamma != 0.0:
        y = y + gamma * prev_ref[...]
    y_ref[...] = y


def _tc_axpy(p, x, prev, alpha, beta, gamma):
    body = functools.partial(_axpy_body, alpha, beta, gamma)
    return pl.pallas_call(
        body,
        out_shape=jax.ShapeDtypeStruct((N0P // 128, 128), F32),
    )(p.reshape(2, N0P // 128, 128), x, prev)


def _wb0(w_ref):
    w4 = w_ref[...][:, 0, :]
    wt = jnp.concatenate([w4, w4, w4, w4], axis=1)
    wt = jnp.concatenate([wt, wt, wt, wt], axis=0)
    ri = lax.broadcasted_iota(jnp.int32, (16, 128), 0)
    ci = lax.broadcasted_iota(jnp.int32, (16, 128), 1)
    return jnp.where(ri // 4 == ci // 32, wt, 0.0)


def _d0p1_body(a_ref, w_ref, b_ref, pool_ref, s1_ref, s2_ref):
    i = pl.program_id(0)
    b128 = jnp.concatenate([b_ref[...]] * 4)
    h = jnp.dot(a_ref[...], _wb0(w_ref), preferred_element_type=F32,
                precision=lax.Precision.HIGHEST) \
        + b128[None, :]
    s1 = jnp.sum(h, axis=0)
    s2 = jnp.sum(h * h, axis=0)

    @pl.when(i == 0)
    def _():
        s1_ref[...] = jnp.zeros((8, 128), F32)
        s2_ref[...] = jnp.zeros((8, 128), F32)

    s1_ref[...] += jnp.broadcast_to(s1[None, :], (8, 128)) * 0.125
    s2_ref[...] += jnp.broadcast_to(s2[None, :], (8, 128)) * 0.125
    pool_ref[...] = jnp.concatenate(
        [jnp.maximum(h[:, 0:32], h[:, 32:64]),
         jnp.maximum(h[:, 64:96], h[:, 96:128])], axis=1)


def _d0p2_body(pool_ref, s1_ref, s2_ref, b_ref, g_ref, be_ref, out_ref):
    b128 = jnp.concatenate([b_ref[...]] * 4)[None, :]
    s1 = jnp.sum(s1_ref[...], axis=0, keepdims=True)
    s2 = jnp.sum(s2_ref[...], axis=0, keepdims=True)
    npad = NX0 - N0 // 4
    m128 = (s1 - npad * b128) / (N0 // 4)
    e128 = (s2 - npad * b128 * b128) / (N0 // 4)
    p = _p32()
    mc = jnp.dot(m128, p, preferred_element_type=F32,
                precision=lax.Precision.HIGHEST)
    e2c = jnp.dot(e128, p, preferred_element_type=F32,
                precision=lax.Precision.HIGHEST)
    sd = jnp.sqrt(e2c - mc * mc + 1e-5)
    g64 = jnp.concatenate([g_ref[...]] * 2)[None, :]
    be64 = jnp.concatenate([be_ref[...]] * 2)[None, :]
    h = g64 * (pool_ref[...] - mc[:, :64]) / sd[:, :64] + be64
    out_ref[...] = _leaky(h)


def _tc_dense0(a, W0, b0, g0, be0):
    grid = NX0 // BK
    pooled, s1, s2 = pl.pallas_call(
        _d0p1_body,
        grid=(grid,),
        in_specs=[pl.BlockSpec((BK, 16), lambda i: (i, 0)),
                  _full_spec((4, 1, 32)), _full_spec((32,))],
        out_specs=[pl.BlockSpec((BK, 64), lambda i: (i, 0)),
                   _acc_spec(), _acc_spec()],
        out_shape=[jax.ShapeDtypeStruct((NX0, 64), F32),
                   jax.ShapeDtypeStruct((8, 128), F32),
                   jax.ShapeDtypeStruct((8, 128), F32)],
    )(a, W0, b0)
    return pl.pallas_call(
        _d0p2_body,
        grid=(grid,),
        in_specs=[pl.BlockSpec((BK, 64), lambda i: (i, 0)),
                  _acc_spec(), _acc_spec(),
                  _full_spec((32,)), _full_spec((32,)), _full_spec((32,))],
        out_specs=pl.BlockSpec((BK, 64), lambda i: (i, 0)),
        out_shape=jax.ShapeDtypeStruct((NX0, 64), F32),
    )(pooled, s1, s2, b0, g0, be0)


def _bd32(w):
    wt = jnp.concatenate([w, w, w, w], axis=1)
    wt = jnp.concatenate([wt, wt, wt, wt], axis=0)
    ri = lax.broadcasted_iota(jnp.int32, (128, 128), 0)
    ci = lax.broadcasted_iota(jnp.int32, (128, 128), 1)
    return jnp.where(ri // 32 == ci // 32, wt, 0.0)


def _mid_h(x_ref, t1_ref, t2_ref, t3_ref, w_ref, b_ref):
    w = w_ref[...]
    h = jnp.dot(x_ref[...], _bd32(w[0]), preferred_element_type=F32,
                precision=lax.Precision.HIGHEST)
    h += jnp.dot(t1_ref[...], _bd32(w[1]), preferred_element_type=F32,
                precision=lax.Precision.HIGHEST)
    h += jnp.dot(t2_ref[...], _bd32(w[2]), preferred_element_type=F32,
                precision=lax.Precision.HIGHEST)
    h += jnp.dot(t3_ref[...], _bd32(w[3]), preferred_element_type=F32,
                precision=lax.Precision.HIGHEST)
    return h + jnp.concatenate([b_ref[...]] * 4)[None, :]


def _midp1_body(x_ref, t1_ref, t2_ref, t3_ref, w_ref, b_ref, s1_ref, s2_ref):
    i = pl.program_id(0)
    h = _mid_h(x_ref, t1_ref, t2_ref, t3_ref, w_ref, b_ref)

    @pl.when(i == 0)
    def _():
        s1_ref[...] = jnp.zeros((8, 128), F32)
        s2_ref[...] = jnp.zeros((8, 128), F32)

    s1_ref[...] += jnp.broadcast_to(jnp.sum(h, axis=0)[None, :], (8, 128)) * 0.125
    s2_ref[...] += jnp.broadcast_to(jnp.sum(h * h, axis=0)[None, :], (8, 128)) * 0.125


def _midp2_body(x_ref, t1_ref, t2_ref, t3_ref, w_ref, b_ref,
                s1_ref, s2_ref, g_ref, be_ref, out_ref):
    h = _mid_h(x_ref, t1_ref, t2_ref, t3_ref, w_ref, b_ref)
    b128 = jnp.concatenate([b_ref[...]] * 4)[None, :]
    s1 = jnp.sum(s1_ref[...], axis=0, keepdims=True)
    s2 = jnp.sum(s2_ref[...], axis=0, keepdims=True)
    npad = NX1 - N1 // 4
    m128 = (s1 - npad * b128) / (N1 // 4)
    e128 = (s2 - npad * b128 * b128) / (N1 // 4)
    p = _p32()
    mc = jnp.dot(m128, p, preferred_element_type=F32,
                precision=lax.Precision.HIGHEST)
    e2c = jnp.dot(e128, p, preferred_element_type=F32,
                precision=lax.Precision.HIGHEST)
    sd = jnp.sqrt(e2c - mc * mc + 1e-5)
    g128 = jnp.concatenate([g_ref[...]] * 4)[None, :]
    be128 = jnp.concatenate([be_ref[...]] * 4)[None, :]
    out = _leaky(g128 * (h - mc) / sd + be128)
    row = pl.program_id(0) * BK + lax.broadcasted_iota(jnp.int32, out.shape, 0)
    out_ref[...] = jnp.where(row < N1 // 4, out, 0.0)


def _tc_dense_mid(x, t1, t2, t3, W1, b1, g1, be1):
    grid = NX1 // BK
    blk = pl.BlockSpec((BK, 128), lambda i: (i, 0))
    s1, s2 = pl.pallas_call(
        _midp1_body,
        grid=(grid,),
        in_specs=[blk, blk, blk, blk,
                  _full_spec((4, 32, 32)), _full_spec((32,))],
        out_specs=[_acc_spec(), _acc_spec()],
        out_shape=[jax.ShapeDtypeStruct((8, 128), F32),
                   jax.ShapeDtypeStruct((8, 128), F32)],
    )(x, t1, t2, t3, W1, b1)
    return pl.pallas_call(
        _midp2_body,
        grid=(grid,),
        in_specs=[blk, blk, blk, blk,
                  _full_spec((4, 32, 32)), _full_spec((32,)),
                  _acc_spec(), _acc_spec(),
                  _full_spec((32,)), _full_spec((32,))],
        out_specs=blk,
        out_shape=jax.ShapeDtypeStruct((NX1, 128), F32),
    )(x, t1, t2, t3, W1, b1, s1, s2, g1, be1)


def _tail_v(x_ref, u1_ref, u2_ref, u3_ref, w_ref, b_ref):
    w = w_ref[...]

    def t128(k):
        return jnp.concatenate([w[k, :, 0]] * 4)[None, :]

    q = (x_ref[...] * t128(0) + u1_ref[...] * t128(1)
         + u2_ref[...] * t128(2) + u3_ref[...] * t128(3))
    ri = lax.broadcasted_iota(jnp.int32, (128, 4), 0)
    ci = lax.broadcasted_iota(jnp.int32, (128, 4), 1)
    sel = jnp.where(ri // 32 == ci, 1.0, 0.0).astype(F32)
    return jnp.dot(q, sel, preferred_element_type=F32,
                precision=lax.Precision.HIGHEST) + b_ref[...][0]


def _tailp1_body(x_ref, u1_ref, u2_ref, u3_ref, w_ref, b_ref,
                 v_ref, s1_ref, s2_ref):
    i = pl.program_id(0)
    v = _tail_v(x_ref, u1_ref, u2_ref, u3_ref, w_ref, b_ref)

    @pl.when(i == 0)
    def _():
        s1_ref[...] = jnp.zeros((8, 128), F32)
        s2_ref[...] = jnp.zeros((8, 128), F32)

    s1_ref[...] += jnp.full((8, 128), jnp.sum(v) / 1024.0, F32)
    s2_ref[...] += jnp.full((8, 128), jnp.sum(v * v) / 1024.0, F32)
    v_ref[...] = v


def _tc_tail_p1(x, u1, u2, u3, W2, b2):
    grid = NX1 // BK
    blk = pl.BlockSpec((BK, 128), lambda i: (i, 0))
    return pl.pallas_call(
        _tailp1_body,
        grid=(grid,),
        in_specs=[blk, blk, blk, blk,
                  _full_spec((4, 32, 1)), _full_spec((1,))],
        out_specs=[pl.BlockSpec((BK, 4), lambda i: (i, 0)),
                   _acc_spec(), _acc_spec()],
        out_shape=[jax.ShapeDtypeStruct((NX1, 4), F32),
                   jax.ShapeDtypeStruct((8, 128), F32),
                   jax.ShapeDtypeStruct((8, 128), F32)],
    )(x, u1, u2, u3, W2, b2)


def _mlp_body(z_ref, s1_ref, s2_ref, b_ref, g_ref, be_ref,
              l1w_ref, l1b_ref, n1g_ref, n1b_ref,
              l2w_ref, l2b_ref, n2g_ref, n2b_ref,
              l3w_ref, l3b_ref, out_ref):
    npad = (NX1 - N1 // 4) * 4
    b2 = b_ref[...][0]
    sv = jnp.sum(s1_ref[...]) - npad * b2
    sq = jnp.sum(s2_ref[...]) - npad * b2 * b2
    m = sv / N1
    var = sq / N1 - m * m
    z = g_ref[...][0] * (z_ref[...] - m) / jnp.sqrt(var + 1e-5) + be_ref[...][0]
    z = _leaky(z)
    z = jnp.dot(z, l1w_ref[...], preferred_element_type=F32,
                precision=lax.Precision.HIGHEST) + l1b_ref[...][None, :]
    m1 = jnp.mean(z, axis=0)
    d1 = z - m1[None, :]
    v1 = jnp.mean(d1 * d1, axis=0)
    z = n1g_ref[...][None, :] * d1 / jnp.sqrt(v1 + 1e-5)[None, :] + n1b_ref[...][None, :]
    z = jnp.maximum(z, 0.0)
    z = jnp.dot(z, l2w_ref[...], preferred_element_type=F32,
                precision=lax.Precision.HIGHEST) + l2b_ref[...][None, :]
    m2 = jnp.mean(z, axis=0)
    d2 = z - m2[None, :]
    v2 = jnp.mean(d2 * d2, axis=0)
    z = n2g_ref[...][None, :] * d2 / jnp.sqrt(v2 + 1e-5)[None, :] + n2b_ref[...][None, :]
    z = jnp.maximum(z, 0.0)
    out_ref[...] = jnp.dot(z, l3w_ref[...], preferred_element_type=F32,
                precision=lax.Precision.HIGHEST) \
        + l3b_ref[...][None, :]


def _tc_mlp(z, s1, s2, b2, g2, be2, lin1_W, lin1_b, bn1_g, bn1_b,
            lin2_W, lin2_b, bn2_g, bn2_b, lin3_W, lin3_b):
    return pl.pallas_call(
        _mlp_body,
        out_shape=jax.ShapeDtypeStruct((16, 1), F32),
    )(z, s1, s2, b2, g2, be2, lin1_W, lin1_b, bn1_g, bn1_b,
      lin2_W, lin2_b, bn2_g, bn2_b, lin3_W, lin3_b)


def _prep_edges(ei, ew, nt):
    cap = nt * ROWS * 128
    src = jnp.pad(ei[0].astype(jnp.int32), (0, cap - ew.shape[0]))
    dst = jnp.pad(ei[1].astype(jnp.int32), (0, cap - ew.shape[0]))
    w = jnp.pad(ew, (0, cap - ew.shape[0]))
    return (src.reshape(nt, ROWS, 128), dst.reshape(nt, ROWS, 128),
            w.reshape(nt, ROWS, 128))


def _to_sc(flat):
    halves = jnp.stack([flat[:, :16], flat[:, 16:]])
    return jnp.pad(halves, ((0, 0), (0, N1P - N1), (0, 0)))


def _to_x4(sc):
    flat = jnp.concatenate([sc[0, :N1], sc[1, :N1]], axis=1)
    return jnp.pad(flat.reshape(N1 // 4, 128), ((0, NX1 - N1 // 4), (0, 0)))


def kernel(x_s, edge_index_s, edge_weight_s, edge_index_s1, edge_weight_s1,
           W0, b0, g0, be0, W1, b1, g1, be1, W2, b2, g2, be2,
           lin1_W, lin1_b, bn1_g, bn1_b, lin2_W, lin2_b, bn2_g, bn2_b,
           lin3_W, lin3_b):
    src0, dst0, ew0 = _prep_edges(edge_index_s, edge_weight_s, NCORE * NSUB)
    src1, dst1, ew1 = _prep_edges(edge_index_s1, edge_weight_s1, NSUB)

    x0f = jnp.pad(x_s[:, 0], (0, N0P - N0))
    x0p = x0f.reshape(N0P // 128, 128)

    p1 = _SC_PROP_C1(src0, dst0, ew0, x0f)
    tx1 = _tc_axpy(p1, x0p, x0p, 1.0, -1.0, 0.0)
    p2 = _SC_PROP_C1(src0, dst0, ew0, tx1.reshape(N0P))
    tx2 = _tc_axpy(p2, tx1, x0p, 2.0, -2.0, -1.0)
    p3 = _SC_PROP_C1(src0, dst0, ew0, tx2.reshape(N0P))
    tx3 = _tc_axpy(p3, tx2, tx1, 2.0, -2.0, -1.0)

    stack = jnp.stack([x0f[:N0], tx1.reshape(N0P)[:N0],
                       tx2.reshape(N0P)[:N0], tx3.reshape(N0P)[:N0]], axis=1)
    a = jnp.pad(stack.reshape(N0 // 4, 16), ((0, NX0 - N0 // 4), (0, 0)))
    pooled = _tc_dense0(a, W0, b0, g0, be0)

    x1_flat = pooled[:N0 // 4].reshape(N1, 32)
    x1_sc = _to_sc(x1_flat)
    x1_x4 = jnp.pad(x1_flat.reshape(N1 // 4, 128), ((0, NX1 - N1 // 4), (0, 0)))

    t1 = _SC_PROP_C32_A(src1, dst1, ew1, x1_sc, x1_sc)
    t2 = _SC_PROP_C32_B(src1, dst1, ew1, t1, x1_sc)
    t3 = _SC_PROP_C32_B(src1, dst1, ew1, t2, t1)
    x2_x4 = _tc_dense_mid(x1_x4, _to_x4(t1), _to_x4(t2), _to_x4(t3),
                          W1, b1, g1, be1)
    x2_sc = _to_sc(x2_x4[:N1 // 4].reshape(N1, 32))

    u1 = _SC_PROP_C32_A(src1, dst1, ew1, x2_sc, x2_sc)
    u2 = _SC_PROP_C32_B(src1, dst1, ew1, u1, x2_sc)
    u3 = _SC_PROP_C32_B(src1, dst1, ew1, u2, u1)

    v4, s1, s2 = _tc_tail_p1(x2_x4, _to_x4(u1), _to_x4(u2), _to_x4(u3), W2, b2)
    z = v4[:N1 // 4].reshape(16, EDGE_NUM // 2)
    return _tc_mlp(z, s1, s2, b2, g2, be2, lin1_W, lin1_b, bn1_g, bn1_b,
                   lin2_W, lin2_b, bn2_g, bn2_b, lin3_W, lin3_b)
